# Initial kernel scaffold; baseline (speedup 1.0000x reference)
import functools

import jax
import jax.numpy as jnp
from jax import lax
from jax.experimental import pallas as pl
from jax.experimental.pallas import tpu as pltpu
from jax.experimental.pallas import tpu_sc as plsc

N_NODES = 10000
N_EDGES = 320000
N_GRAPHS = 256

NC, NS = 2, 16
NW = NC * NS
N_PAD = 10240
EB = 128
NB = 79
EPW = NB * EB
NE_PAD = EPW * NW

_MESH = plsc.VectorSubcoreMesh(
    core_axis_name="c", subcore_axis_name="s", num_cores=NC, num_subcores=NS
)


def _seg_sum_call(src, dst, h):
    D = h.shape[1]
    rows_per_tile = N_PAD // NS
    ZR = 64

    @functools.partial(
        pl.kernel,
        out_type=jax.ShapeDtypeStruct((NC, N_PAD, D), jnp.float32),
        mesh=_MESH,
        scratch_types=[
            pltpu.VMEM((NB, EB), jnp.int32),
            pltpu.VMEM((NB, EB), jnp.int32),
            pltpu.VMEM((EB, D), jnp.float32),
            pltpu.VMEM((ZR, D), jnp.float32),
            pltpu.VMEM_SHARED((N_PAD, D), jnp.float32),
            pltpu.SemaphoreType.DMA,
        ],
    )
    def k(src_hbm, dst_hbm, h_hbm, out_hbm, src_v, dst_v, rows_v, zero_v,
          acc_sh, sem):
        cid = lax.axis_index("c")
        sid = lax.axis_index("s")
        wid = cid * NS + sid

        pltpu.sync_copy(src_hbm.at[wid], src_v)
        pltpu.sync_copy(dst_hbm.at[wid], dst_v)

        zeros16 = jnp.zeros((16,), jnp.float32)

        @pl.loop(0, ZR)
        def _(r):
            @pl.loop(0, D, step=16)
            def _(cc):
                zero_v[r, pl.ds(cc, 16)] = zeros16

        @pl.loop(0, rows_per_tile, step=ZR)
        def _(j):
            pltpu.sync_copy(zero_v, acc_sh.at[pl.ds(sid * rows_per_tile + j, ZR)])

        plsc.subcore_barrier()

        @pl.loop(0, NB)
        def _(b):
            pltpu.async_copy(h_hbm.at[src_v.at[b]], rows_v, sem).wait()
            pltpu.sync_copy(rows_v, acc_sh.at[dst_v.at[b]], add=True)

        plsc.subcore_barrier()

        r0 = sid * rows_per_tile
        pltpu.sync_copy(acc_sh.at[pl.ds(r0, rows_per_tile)],
                        out_hbm.at[cid, pl.ds(r0, rows_per_tile)])

    return k(src, dst, h)


def kernel(text, edge_index, graph_ids, emb, W1, b1, W2, b2, Wc, bc):
    text = text.astype(jnp.int32)
    graph_ids = graph_ids.astype(jnp.int32)
    src = edge_index[0].astype(jnp.int32)
    dst = edge_index[1].astype(jnp.int32)

    pad_e = NE_PAD - N_EDGES
    src_p = jnp.concatenate([src, jnp.full((pad_e,), N_NODES, jnp.int32)])
    dst_p = jnp.concatenate([dst, jnp.full((pad_e,), N_NODES, jnp.int32)])
    src_p = src_p.reshape(NW, NB, EB)
    dst_p = dst_p.reshape(NW, NB, EB)

    deg_out = jnp.clip(jnp.bincount(src, length=N_NODES), 1).astype(jnp.float32)
    deg_in = jnp.clip(jnp.bincount(dst, length=N_NODES), 1).astype(jnp.float32)
    s_out = deg_out ** -0.5
    s_in = deg_in ** -0.5

    h0 = emb[text]

    def conv(h, W, b):
        D = h.shape[1]
        hs = h * s_out[:, None]
        hs_pad = jnp.concatenate(
            [hs, jnp.zeros((N_PAD - N_NODES, D), jnp.float32)])
        parts = _seg_sum_call(src_p, dst_p, hs_pad)
        agg = (parts[0] + parts[1])[:N_NODES]
        agg = agg * s_in[:, None]
        return agg @ W + b

    h1 = jax.nn.relu(conv(h0, W1, b1))
    h2 = jax.nn.relu(conv(h1, W2, b2))
    hg = jax.ops.segment_max(h2, graph_ids, num_segments=N_GRAPHS)
    hg = jnp.maximum(hg, 0.0)
    return hg @ Wc + bc


# speedup vs baseline: 4.4355x; 4.4355x over previous
import functools

import jax
import jax.numpy as jnp
from jax import lax
from jax.experimental import pallas as pl
from jax.experimental.pallas import tpu as pltpu
from jax.experimental.pallas import tpu_sc as plsc

N_NODES = 10000
N_EDGES = 320000
N_GRAPHS = 256

NC, NS = 2, 16
NW = NC * NS
N_PAD = 10240
EB = 128
NB = 79
EPW = NB * EB
NE_PAD = EPW * NW

_MESH = plsc.VectorSubcoreMesh(
    core_axis_name="c", subcore_axis_name="s", num_cores=NC, num_subcores=NS
)


def _seg_sum_call(src, dst, h):
    D = h.shape[1]
    rows_per_tile = N_PAD // NS
    ZR = 64

    @functools.partial(
        pl.kernel,
        out_type=jax.ShapeDtypeStruct((NC, N_PAD, D), jnp.float32),
        mesh=_MESH,
        scratch_types=[
            pltpu.VMEM((NB, EB), jnp.int32),
            pltpu.VMEM((NB, EB), jnp.int32),
            pltpu.VMEM((EB, D), jnp.float32),
            pltpu.VMEM((ZR, D), jnp.float32),
            pltpu.VMEM_SHARED((N_PAD, D), jnp.float32),
            pltpu.SemaphoreType.DMA,
        ],
        compiler_params=pltpu.CompilerParams(use_tc_tiling_on_sc=False),
    )
    def k(src_hbm, dst_hbm, h_hbm, out_hbm, src_v, dst_v, rows_v, zero_v,
          acc_sh, sem):
        cid = lax.axis_index("c")
        sid = lax.axis_index("s")
        wid = cid * NS + sid

        pltpu.sync_copy(src_hbm.at[wid], src_v)
        pltpu.sync_copy(dst_hbm.at[wid], dst_v)

        zeros16 = jnp.zeros((16,), jnp.float32)

        @pl.loop(0, ZR)
        def _(r):
            @pl.loop(0, D, step=16)
            def _(cc):
                zero_v[r, pl.ds(cc, 16)] = zeros16

        @pl.loop(0, rows_per_tile, step=ZR)
        def _(j):
            pltpu.sync_copy(zero_v, acc_sh.at[pl.ds(sid * rows_per_tile + j, ZR)])

        plsc.subcore_barrier()

        @pl.loop(0, NB)
        def _(b):
            pltpu.async_copy(h_hbm.at[src_v.at[b]], rows_v, sem).wait()
            pltpu.sync_copy(rows_v, acc_sh.at[dst_v.at[b]], add=True)

        plsc.subcore_barrier()

        r0 = sid * rows_per_tile
        pltpu.sync_copy(acc_sh.at[pl.ds(r0, rows_per_tile)],
                        out_hbm.at[cid, pl.ds(r0, rows_per_tile)])

    return k(src, dst, h)


def kernel(text, edge_index, graph_ids, emb, W1, b1, W2, b2, Wc, bc):
    text = text.astype(jnp.int32)
    graph_ids = graph_ids.astype(jnp.int32)
    src = edge_index[0].astype(jnp.int32)
    dst = edge_index[1].astype(jnp.int32)

    pad_e = NE_PAD - N_EDGES
    src_p = jnp.concatenate([src, jnp.full((pad_e,), N_NODES, jnp.int32)])
    dst_p = jnp.concatenate([dst, jnp.full((pad_e,), N_NODES, jnp.int32)])
    src_p = src_p.reshape(NW, NB, EB)
    dst_p = dst_p.reshape(NW, NB, EB)

    deg_out = jnp.clip(jnp.bincount(src, length=N_NODES), 1).astype(jnp.float32)
    deg_in = jnp.clip(jnp.bincount(dst, length=N_NODES), 1).astype(jnp.float32)
    s_out = deg_out ** -0.5
    s_in = deg_in ** -0.5

    h0 = emb[text]

    def conv(h, W, b):
        D = h.shape[1]
        hs = h * s_out[:, None]
        hs_pad = jnp.concatenate(
            [hs, jnp.zeros((N_PAD - N_NODES, D), jnp.float32)])
        parts = _seg_sum_call(src_p, dst_p, hs_pad)
        agg = (parts[0] + parts[1])[:N_NODES]
        agg = agg * s_in[:, None]
        return agg @ W + b

    h1 = jax.nn.relu(conv(h0, W1, b1))
    h2 = jax.nn.relu(conv(h1, W2, b2))
    hg = jax.ops.segment_max(h2, graph_ids, num_segments=N_GRAPHS)
    hg = jnp.maximum(hg, 0.0)
    return hg @ Wc + bc
